# Initial kernel scaffold; baseline (speedup 1.0000x reference)
import jax
import jax.numpy as jnp
from jax.experimental import pallas as pl


def kernel(obs, mask, W_embed, b_embed, Wp, bp, W_ih, W_hh, b_ih, b_hh, W_out, b_out):
    raise NotImplementedError("write your pallas kernel here")


# speedup vs baseline: 12.9965x; 12.9965x over previous
import jax
import jax.numpy as jnp
from jax import lax
from jax.experimental import pallas as pl
from jax.experimental.pallas import tpu as pltpu

EMB = 64
HID = 128
G = 8
NB = 32.0
CELL = 2.0 * NB / G
PRED = 12

_INTERPRET = False


def _dg(x, w):
    return lax.dot_general(x, w, (((1,), (1,)), ((), ())),
                           preferred_element_type=jnp.float32)


def _dgb(x, w):
    return lax.dot_general(x.astype(jnp.bfloat16), w,
                           (((1,), (1,)), ((), ())),
                           preferred_element_type=jnp.float32)


def _kernel_body(obs_ref, obsT_ref, mask2_ref, maskH_ref, maskT_ref,
                 W_embed_ref, b_embed_ref, WpT_ref, bp_ref,
                 Wih_ref, Whh_ref, bih_ref, bhh_ref,
                 Wout5_ref, bout5_ref,
                 mus_ref, sig_ref, rho_ref, gf_ref):
    f32 = jnp.float32
    t_obs = obs_ref.shape[0]
    n = obs_ref.shape[1]

    mask2 = mask2_ref[...]
    maskH = maskH_ref[...]
    maskT = maskT_ref[...]
    rows = lax.broadcasted_iota(jnp.int32, (n, n), 0)
    cols = lax.broadcasted_iota(jnp.int32, (n, n), 1)
    neye = (rows != cols).astype(f32)
    pairm = maskH[:, 0:1] * maskT * neye

    b_embed = b_embed_ref[...]
    bp = bp_ref[...]
    bih = bih_ref[...]
    bhh = bhh_ref[...]
    W_embed = W_embed_ref[...]
    Wih = Wih_ref[...]
    Whh = Whh_ref[...]
    Wout5 = Wout5_ref[...]
    bout5 = bout5_ref[...]

    def social(h, pxc, pxr, pyc, pyr):
        rx = pxr - pxc
        ry = pyr - pyc
        colf = jnp.floor((rx + NB) / CELL)
        rowf = jnp.floor((ry + NB) / CELL)
        validf = (pairm
                  * (jnp.abs(rx) < NB).astype(f32)
                  * (jnp.abs(ry) < NB).astype(f32))
        cellf = rowf * G + colf

        bf = jnp.bfloat16
        h_hi = h.astype(bf)
        r1 = h - h_hi.astype(f32)
        h_mid = r1.astype(bf)
        h_lo = (r1 - h_mid.astype(f32)).astype(bf)

        def cell_body(c, _):
            Mc = (validf * (cellf == c.astype(f32)).astype(f32)).astype(bf)
            grid_c = (jnp.dot(Mc, h_hi, preferred_element_type=f32)
                      + jnp.dot(Mc, h_mid, preferred_element_type=f32)
                      + jnp.dot(Mc, h_lo, preferred_element_type=f32))
            gf_ref[:, pl.ds(c * HID, HID)] = grid_c
            return 0

        lax.fori_loop(0, G * G, cell_body, 0)
        pooled = jnp.dot(gf_ref[...].astype(jnp.bfloat16), WpT_ref[...],
                         preferred_element_type=f32)
        return (pooled + bp) * maskH

    def lstm(emb, soc, h, c):
        inp = jnp.concatenate([emb, soc], axis=-1)
        gates = _dgb(inp, Wih) + bih + _dgb(h, Whh) + bhh
        i = jax.nn.sigmoid(gates[:, 0 * HID:1 * HID])
        f = jax.nn.sigmoid(gates[:, 1 * HID:2 * HID])
        g = jnp.tanh(gates[:, 2 * HID:3 * HID])
        o = jax.nn.sigmoid(gates[:, 3 * HID:4 * HID])
        c2 = f * c + i * g
        h2 = o * jnp.tanh(c2)
        return h2, c2

    def obs_body(t, hc):
        h, c = hc
        pos = obs_ref[t]
        pos = jnp.where(jnp.isnan(pos), 0.0, pos)
        posT = obsT_ref[t]
        posT = jnp.where(jnp.isnan(posT), 0.0, posT)
        emb = jax.nn.relu(_dgb(pos, W_embed) + b_embed)
        soc = social(h, pos[:, 0:1], posT[0:1, :], pos[:, 1:2], posT[1:2, :])
        return lstm(emb, soc, h, c)

    h0 = jnp.zeros((n, HID), f32)
    h, c = lax.fori_loop(0, t_obs, obs_body, (h0, h0))

    pos_last = obs_ref[t_obs - 1]
    pos_last = jnp.where(jnp.isnan(pos_last), 0.0, pos_last)
    posT_last = obsT_ref[t_obs - 1]
    posT_last = jnp.where(jnp.isnan(posT_last), 0.0, posT_last)
    cur = pos_last * mask2
    curT = posT_last * maskT

    def pred_body(t, carry):
        h, c, cur, curT = carry
        emb = jax.nn.relu(_dgb(cur, W_embed) + b_embed)
        soc = social(h, cur[:, 0:1], curT[0:1, :], cur[:, 1:2], curT[1:2, :])
        h, c = lstm(emb, soc, h, c)
        raw5 = _dgb(h, Wout5) + bout5
        mu = raw5[:, 0:2]
        sig = jnp.exp(raw5[:, 2:4]) + 1e-6
        rho = jnp.tanh(raw5[:, 4:5])
        dlt = mu * mask2
        cur = cur + dlt
        curT = curT + jnp.transpose(dlt, (1, 0))
        mus_ref[t] = cur
        sig_ref[t] = sig
        rho_ref[t] = rho
        return h, c, cur, curT

    lax.fori_loop(0, PRED, pred_body, (h, c, cur, curT))


def kernel(obs, mask, W_embed, b_embed, Wp, bp, W_ih, W_hh, b_ih, b_hh,
           W_out, b_out):
    t_obs, n, _ = obs.shape
    f32 = jnp.float32

    maskf = mask.astype(f32)
    mask2 = jnp.broadcast_to(maskf.reshape(n, 1), (n, 2))
    maskH = jnp.broadcast_to(maskf.reshape(n, 1), (n, HID))
    maskT = maskf.reshape(1, n)
    obsT = jnp.transpose(obs, (0, 2, 1))

    bf16 = jnp.bfloat16
    WpT = jnp.transpose(Wp).astype(bf16)

    args = (
        obs, obsT, mask2, maskH, maskT,
        W_embed.astype(bf16), b_embed.reshape(1, EMB), WpT, bp.reshape(1, HID),
        W_ih.astype(bf16), W_hh.astype(bf16),
        b_ih.reshape(1, 4 * HID), b_hh.reshape(1, 4 * HID),
        W_out.astype(bf16), b_out.reshape(1, 5),
    )
    out_shape = (
        jax.ShapeDtypeStruct((PRED, n, 2), f32),
        jax.ShapeDtypeStruct((PRED, n, 2), f32),
        jax.ShapeDtypeStruct((PRED, n, 1), f32),
    )
    mus, sigmas, rhos3 = pl.pallas_call(
        _kernel_body,
        out_shape=out_shape,
        scratch_shapes=[pltpu.VMEM((n, G * G * HID), f32)],
        interpret=_INTERPRET,
    )(*args)
    return mus, sigmas, rhos3[..., 0]
